# Initial kernel scaffold; baseline (speedup 1.0000x reference)
import jax
import jax.numpy as jnp
from jax.experimental import pallas as pl


def kernel(features, A, W, att_src, att_dst, bias):
    raise NotImplementedError("write your pallas kernel here")


# speedup vs baseline: 4473.1974x; 4473.1974x over previous
import functools

import jax
import jax.numpy as jnp
from jax.experimental import pallas as pl
from jax.experimental.pallas import tpu as pltpu

IN_DIM = 32
OUT_DIM = 32
HEADS = 4
OUT_CH = OUT_DIM // HEADS
B = 4
N = 1024
TJ = 256


def _gat_tile_kernel(feat_ref, featd_ref, a_ref, w_ref, asrc_ref, adst_ref,
                     bias_ref, out_ref):
    j0 = pl.program_id(0) * TJ

    a_blk = a_ref[...]
    row_i = jax.lax.broadcasted_iota(jnp.int32, (N, TJ), 0)
    col_j = jax.lax.broadcasted_iota(jnp.int32, (N, TJ), 1) + j0
    cnt = (a_blk != 0).astype(jnp.float32) + (row_i == col_j).astype(
        jnp.float32)
    has = cnt > 0.0

    w = w_ref[...]
    a_s = asrc_ref[...]
    a_d = adst_ref[...]
    bias = bias_ref[...]

    for b in range(B):
        xb = feat_ref[b]
        hb = jnp.dot(xb, w, preferred_element_type=jnp.float32,
                     precision=jax.lax.Precision.HIGHEST)
        src_l = jnp.dot(hb, a_s, preferred_element_type=jnp.float32,
                        precision=jax.lax.Precision.HIGHEST)
        hb_tile = jnp.dot(featd_ref[b], w, preferred_element_type=jnp.float32,
                          precision=jax.lax.Precision.HIGHEST)
        dst_l = jax.lax.dot_general(
            a_d, hb_tile, (((0,), (1,)), ((), ())),
            preferred_element_type=jnp.float32,
            precision=jax.lax.Precision.HIGHEST)

        head_outs = []
        for h in range(HEADS):
            z = src_l[:, h:h + 1] + dst_l[h:h + 1, :]
            s = jnp.where(z >= 0.0, z, 0.2 * z)
            m = jnp.max(jnp.where(has, s, -jnp.inf), axis=0, keepdims=True)
            p = cnt * jnp.where(has, jnp.exp(s - m), 0.0)
            denom = jnp.sum(p, axis=0, keepdims=True)
            pn = p * (1.0 / jnp.maximum(denom, 1e-16))
            oh = jax.lax.dot_general(
                pn, hb[:, h * OUT_CH:(h + 1) * OUT_CH],
                (((0,), (0,)), ((), ())),
                preferred_element_type=jnp.float32,
                precision=jax.lax.Precision.HIGHEST)
            head_outs.append(oh)
        out_ref[b] = jnp.concatenate(head_outs, axis=1) + bias


@functools.partial(jax.jit, static_argnames=())
def kernel(features, A, W, att_src, att_dst, bias):
    eye = jnp.eye(HEADS, dtype=jnp.float32)
    asrc_mat = (att_src[:, :, None] * eye[:, None, :]).reshape(
        HEADS * OUT_CH, HEADS)
    adst_mat = (att_dst[:, :, None] * eye[:, None, :]).reshape(
        HEADS * OUT_CH, HEADS)
    bias2d = bias.reshape(1, HEADS * OUT_CH)

    grid = (N // TJ,)
    out = pl.pallas_call(
        _gat_tile_kernel,
        grid=grid,
        in_specs=[
            pl.BlockSpec((B, N, IN_DIM), lambda j: (0, 0, 0)),
            pl.BlockSpec((B, TJ, IN_DIM), lambda j: (0, j, 0)),
            pl.BlockSpec((N, TJ), lambda j: (0, j)),
            pl.BlockSpec((IN_DIM, HEADS * OUT_CH), lambda j: (0, 0)),
            pl.BlockSpec((HEADS * OUT_CH, HEADS), lambda j: (0, 0)),
            pl.BlockSpec((HEADS * OUT_CH, HEADS), lambda j: (0, 0)),
            pl.BlockSpec((1, HEADS * OUT_CH), lambda j: (0, 0)),
        ],
        out_specs=pl.BlockSpec((B, TJ, HEADS * OUT_CH), lambda j: (0, j, 0)),
        out_shape=jax.ShapeDtypeStruct((B, N, HEADS * OUT_CH), jnp.float32),
        compiler_params=pltpu.CompilerParams(
            dimension_semantics=("arbitrary",)),
    )(features, features, A, W, asrc_mat, adst_mat, bias2d)
    return out


# speedup vs baseline: 5358.2854x; 1.1979x over previous
import functools

import jax
import jax.numpy as jnp
from jax.experimental import pallas as pl
from jax.experimental.pallas import tpu as pltpu

IN_DIM = 32
OUT_DIM = 32
HEADS = 4
OUT_CH = OUT_DIM // HEADS
B = 4
N = 1024
TJ = 256


def _gat_tile_kernel(feat_ref, featd_ref, a_ref, w_ref, asrc_ref, adst_ref,
                     bias_ref, out_ref):
    j0 = pl.program_id(0) * TJ

    a_blk = a_ref[...]
    row_i = jax.lax.broadcasted_iota(jnp.int32, (N, TJ), 0)
    col_j = jax.lax.broadcasted_iota(jnp.int32, (N, TJ), 1) + j0
    cnt = (a_blk != 0).astype(jnp.float32) + (row_i == col_j).astype(
        jnp.float32)

    w = w_ref[...]
    a_s = asrc_ref[...]
    a_d = adst_ref[...]
    bias = bias_ref[...]

    ones_col = jnp.ones((N, 1), dtype=jnp.float32)
    for b in range(B):
        xb = feat_ref[b]
        hb = jnp.dot(xb, w, preferred_element_type=jnp.float32,
                     precision=jax.lax.Precision.HIGHEST)
        src_l = jnp.dot(hb, a_s, preferred_element_type=jnp.float32,
                        precision=jax.lax.Precision.HIGHEST)
        hb_tile = jnp.dot(featd_ref[b], w, preferred_element_type=jnp.float32,
                          precision=jax.lax.Precision.HIGHEST)
        dst_l = jax.lax.dot_general(
            a_d, hb_tile, (((0,), (1,)), ((), ())),
            preferred_element_type=jnp.float32,
            precision=jax.lax.Precision.HIGHEST)
        src_max = jnp.max(src_l, axis=0, keepdims=True)

        head_outs = []
        for h in range(HEADS):
            zm = src_max[0, h] + dst_l[h:h + 1, :]
            m = jnp.maximum(zm, 0.2 * zm)
            z = src_l[:, h:h + 1] + dst_l[h:h + 1, :]
            s = jnp.maximum(z, 0.2 * z)
            p = cnt * jnp.exp(s - m)
            rhs = jnp.concatenate(
                [hb[:, h * OUT_CH:(h + 1) * OUT_CH], ones_col], axis=1)
            agg = jax.lax.dot_general(
                p, rhs, (((0,), (0,)), ((), ())),
                preferred_element_type=jnp.float32,
                precision=jax.lax.Precision.HIGHEST)
            denom = jnp.maximum(agg[:, OUT_CH:OUT_CH + 1], 1e-16)
            head_outs.append(agg[:, :OUT_CH] * (1.0 / denom))
        out_ref[b] = jnp.concatenate(head_outs, axis=1) + bias


@functools.partial(jax.jit, static_argnames=())
def kernel(features, A, W, att_src, att_dst, bias):
    eye = jnp.eye(HEADS, dtype=jnp.float32)
    asrc_mat = (att_src[:, :, None] * eye[:, None, :]).reshape(
        HEADS * OUT_CH, HEADS)
    adst_mat = (att_dst[:, :, None] * eye[:, None, :]).reshape(
        HEADS * OUT_CH, HEADS)
    bias2d = bias.reshape(1, HEADS * OUT_CH)

    grid = (N // TJ,)
    out = pl.pallas_call(
        _gat_tile_kernel,
        grid=grid,
        in_specs=[
            pl.BlockSpec((B, N, IN_DIM), lambda j: (0, 0, 0)),
            pl.BlockSpec((B, TJ, IN_DIM), lambda j: (0, j, 0)),
            pl.BlockSpec((N, TJ), lambda j: (0, j)),
            pl.BlockSpec((IN_DIM, HEADS * OUT_CH), lambda j: (0, 0)),
            pl.BlockSpec((HEADS * OUT_CH, HEADS), lambda j: (0, 0)),
            pl.BlockSpec((HEADS * OUT_CH, HEADS), lambda j: (0, 0)),
            pl.BlockSpec((1, HEADS * OUT_CH), lambda j: (0, 0)),
        ],
        out_specs=pl.BlockSpec((B, TJ, HEADS * OUT_CH), lambda j: (0, j, 0)),
        out_shape=jax.ShapeDtypeStruct((B, N, HEADS * OUT_CH), jnp.float32),
        compiler_params=pltpu.CompilerParams(
            dimension_semantics=("parallel",)),
    )(features, features, A, W, asrc_mat, adst_mat, bias2d)
    return out


# speedup vs baseline: 6721.9384x; 1.2545x over previous
import functools

import jax
import jax.numpy as jnp
from jax.experimental import pallas as pl
from jax.experimental.pallas import tpu as pltpu

IN_DIM = 32
OUT_DIM = 32
HEADS = 4
OUT_CH = OUT_DIM // HEADS
B = 4
N = 1024
TJ = 256


def _gat_tile_kernel(feat_ref, featd_ref, a_ref, w_ref, asrc_ref, adst_ref,
                     bias_ref, out_ref):
    mask = a_ref[...] != 0

    w = w_ref[...]
    a_s = asrc_ref[...]
    a_d = adst_ref[...]
    bias = bias_ref[...]

    ones_col = jnp.ones((N, 1), dtype=jnp.float32)
    for b in range(B):
        xb = feat_ref[b]
        hb = jnp.dot(xb, w, preferred_element_type=jnp.float32,
                     precision=jax.lax.Precision.HIGHEST)
        src_l = jnp.dot(hb, a_s, preferred_element_type=jnp.float32,
                        precision=jax.lax.Precision.HIGHEST)
        hb_tile = jnp.dot(featd_ref[b], w, preferred_element_type=jnp.float32,
                          precision=jax.lax.Precision.HIGHEST)
        dst_l = jax.lax.dot_general(
            a_d, hb_tile, (((0,), (1,)), ((), ())),
            preferred_element_type=jnp.float32,
            precision=jax.lax.Precision.HIGHEST)
        src_l_tile = jnp.dot(hb_tile, a_s, preferred_element_type=jnp.float32,
                             precision=jax.lax.Precision.HIGHEST)
        dst_l_tile = jnp.dot(hb_tile, a_d, preferred_element_type=jnp.float32,
                             precision=jax.lax.Precision.HIGHEST)
        src_max = jnp.max(src_l, axis=0, keepdims=True)

        head_outs = []
        for h in range(HEADS):
            zm = src_max[0, h] + dst_l[h:h + 1, :]
            m = jnp.maximum(zm, 0.2 * zm)
            z = src_l[:, h:h + 1] + dst_l[h:h + 1, :]
            s = jnp.maximum(z, 0.2 * z)
            p = jnp.where(mask, jnp.exp(s - m), 0.0)
            rhs = jnp.concatenate(
                [hb[:, h * OUT_CH:(h + 1) * OUT_CH], ones_col], axis=1)
            agg = jax.lax.dot_general(
                p.astype(jnp.bfloat16), rhs.astype(jnp.bfloat16),
                (((0,), (0,)), ((), ())),
                preferred_element_type=jnp.float32)
            zc = src_l_tile[:, h:h + 1] + dst_l_tile[:, h:h + 1]
            mc = src_max[0, h] + dst_l_tile[:, h:h + 1]
            mc = jnp.maximum(mc, 0.2 * mc)
            ex_d = jnp.exp(jnp.maximum(zc, 0.2 * zc) - mc)
            num = agg[:, :OUT_CH] + ex_d * hb_tile[:, h * OUT_CH:
                                                   (h + 1) * OUT_CH]
            denom = jnp.maximum(agg[:, OUT_CH:OUT_CH + 1] + ex_d, 1e-16)
            head_outs.append(num * (1.0 / denom))
        out_ref[b] = jnp.concatenate(head_outs, axis=1) + bias


@functools.partial(jax.jit, static_argnames=())
def kernel(features, A, W, att_src, att_dst, bias):
    eye = jnp.eye(HEADS, dtype=jnp.float32)
    asrc_mat = (att_src[:, :, None] * eye[:, None, :]).reshape(
        HEADS * OUT_CH, HEADS)
    adst_mat = (att_dst[:, :, None] * eye[:, None, :]).reshape(
        HEADS * OUT_CH, HEADS)
    bias2d = bias.reshape(1, HEADS * OUT_CH)

    grid = (N // TJ,)
    out = pl.pallas_call(
        _gat_tile_kernel,
        grid=grid,
        in_specs=[
            pl.BlockSpec((B, N, IN_DIM), lambda j: (0, 0, 0)),
            pl.BlockSpec((B, TJ, IN_DIM), lambda j: (0, j, 0)),
            pl.BlockSpec((N, TJ), lambda j: (0, j)),
            pl.BlockSpec((IN_DIM, HEADS * OUT_CH), lambda j: (0, 0)),
            pl.BlockSpec((HEADS * OUT_CH, HEADS), lambda j: (0, 0)),
            pl.BlockSpec((HEADS * OUT_CH, HEADS), lambda j: (0, 0)),
            pl.BlockSpec((1, HEADS * OUT_CH), lambda j: (0, 0)),
        ],
        out_specs=pl.BlockSpec((B, TJ, HEADS * OUT_CH), lambda j: (0, j, 0)),
        out_shape=jax.ShapeDtypeStruct((B, N, HEADS * OUT_CH), jnp.float32),
        compiler_params=pltpu.CompilerParams(
            dimension_semantics=("parallel",)),
    )(features, features, A, W, asrc_mat, adst_mat, bias2d)
    return out
